# baseline (speedup 1.0000x reference)
import jax
import jax.numpy as jnp
from jax import lax
from jax.experimental import pallas as pl
from jax.experimental.pallas import tpu as pltpu
from jax.experimental.pallas import tpu_sc as plsc

_NC, _NS, _L = 2, 16, 16
_NW = _NC * _NS


def _make_body(B, N, D, bpw):
    half = bpw // 2

    def body(x_hbm, idx_hbm, out_hbm, idx_a, idx_b, rows_a, rows_b,
             sem_a, sem_b, sem_sa, sem_sb):
        wid = lax.axis_index("s") * _NC + lax.axis_index("c")
        base = wid * bpw
        pltpu.sync_copy(idx_hbm.at[pl.ds(base, half)], idx_a)
        for j in range(half // _L):
            seg = pl.ds(j * _L, _L)
            batch_ids = (base + j * _L) + lax.broadcasted_iota(
                jnp.int32, (_L,), 0
            )
            idx_a[seg] = batch_ids * N + idx_a[seg]
        ga = pltpu.async_copy(x_hbm.at[idx_a], rows_a, sem_a)
        pltpu.sync_copy(idx_hbm.at[pl.ds(base + half, half)], idx_b)
        for j in range(half // _L):
            seg = pl.ds(j * _L, _L)
            batch_ids = (base + half + j * _L) + lax.broadcasted_iota(
                jnp.int32, (_L,), 0
            )
            idx_b[seg] = batch_ids * N + idx_b[seg]
        gb = pltpu.async_copy(x_hbm.at[idx_b], rows_b, sem_b)
        ga.wait()
        sa = pltpu.async_copy(rows_a, out_hbm.at[pl.ds(base, half)], sem_sa)
        gb.wait()
        sb = pltpu.async_copy(
            rows_b, out_hbm.at[pl.ds(base + half, half)], sem_sb
        )
        sa.wait()
        sb.wait()

    return body


def kernel(x, edge_index, agent_nodes):
    del edge_index
    B, N, D = x.shape
    bpw = B // _NW
    x_flat = x.reshape(B * N, D)
    idx32 = agent_nodes.astype(jnp.int32)
    mesh = plsc.VectorSubcoreMesh(core_axis_name="c", subcore_axis_name="s")
    k = pl.kernel(
        _make_body(B, N, D, bpw),
        mesh=mesh,
        out_type=jax.ShapeDtypeStruct((B, D), jnp.float32),
        scratch_types=[
            pltpu.VMEM((bpw // 2,), jnp.int32),
            pltpu.VMEM((bpw // 2,), jnp.int32),
            pltpu.VMEM((bpw // 2, D), jnp.float32),
            pltpu.VMEM((bpw // 2, D), jnp.float32),
            pltpu.SemaphoreType.DMA,
            pltpu.SemaphoreType.DMA,
            pltpu.SemaphoreType.DMA,
            pltpu.SemaphoreType.DMA,
        ],
    )
    return k(x_flat, idx32)


# speedup vs baseline: 1.0903x; 1.0903x over previous
import jax
import jax.numpy as jnp
from jax import lax
from jax.experimental import pallas as pl
from jax.experimental.pallas import tpu as pltpu
from jax.experimental.pallas import tpu_sc as plsc

_NC, _NS, _L = 1, 16, 16
_NW = _NC * _NS


def _make_body(B, N, D, bpw):
    def body(x_hbm, idx_hbm, out_hbm, idx_v, rows_v, sem):
        wid = lax.axis_index("s") * _NC + lax.axis_index("c")
        base = wid * bpw
        pltpu.sync_copy(idx_hbm.at[pl.ds(base, bpw)], idx_v)
        for j in range(bpw // _L):
            seg = pl.ds(j * _L, _L)
            batch_ids = (base + j * _L) + lax.broadcasted_iota(
                jnp.int32, (_L,), 0
            )
            idx_v[seg] = batch_ids * N + idx_v[seg]
        pltpu.async_copy(x_hbm.at[idx_v], rows_v, sem).wait()
        pltpu.sync_copy(rows_v, out_hbm.at[pl.ds(base, bpw)])

    return body


def kernel(x, edge_index, agent_nodes):
    del edge_index
    B, N, D = x.shape
    bpw = B // _NW
    x_flat = x.reshape(B * N, D)
    idx32 = agent_nodes.astype(jnp.int32)
    mesh = plsc.VectorSubcoreMesh(
        core_axis_name="c", subcore_axis_name="s", num_cores=_NC
    )
    k = pl.kernel(
        _make_body(B, N, D, bpw),
        mesh=mesh,
        out_type=jax.ShapeDtypeStruct((B, D), jnp.float32),
        scratch_types=[
            pltpu.VMEM((bpw,), jnp.int32),
            pltpu.VMEM((bpw, D), jnp.float32),
            pltpu.SemaphoreType.DMA,
        ],
    )
    return k(x_flat, idx32)
